# baseline (device time: 23473 ns/iter reference)
import jax
import jax.numpy as jnp
from jax import lax
from jax.experimental import pallas as pl
from jax.experimental.pallas import tpu as pltpu

Z_DEV = 4
B, SQ, SKV, H, D = 8, 1, 512, 8, 64
HD = H * D
PACK = HD + 128


def kernel(Q, K, V):
    k2 = K.reshape(B, SKV, HD)
    v2 = V.reshape(B, SKV, HD)
    q2 = Q.reshape(B, HD)

    def body(q_ref, k_hbm, v_hbm, out_ref, kbuf, vbuf, comm,
             kcp_sems, vcp_sems, send_sems, recv_sems):
        my_x = lax.axis_index("x")
        my_y = lax.axis_index("y")
        my_z = lax.axis_index("z")

        barrier_sem = pltpu.get_barrier_semaphore()
        for r in (1, 2, 3):
            pl.semaphore_signal(
                barrier_sem,
                inc=1,
                device_id=(my_x, my_y, (my_z + r) % Z_DEV),
                device_id_type=pl.DeviceIdType.MESH,
            )

        ids_hd = lax.broadcasted_iota(jnp.int32, (H, HD), 1) // D
        ids_h = lax.broadcasted_iota(jnp.int32, (H, HD), 0)
        e8 = (ids_hd == ids_h).astype(jnp.float32)
        eye8 = (
            lax.broadcasted_iota(jnp.int32, (H, H), 0)
            == lax.broadcasted_iota(jnp.int32, (H, H), 1)
        ).astype(jnp.float32)

        def copy_in(b, slot):
            kc = pltpu.make_async_copy(k_hbm.at[b], kbuf.at[slot],
                                       kcp_sems.at[slot])
            vc = pltpu.make_async_copy(v_hbm.at[b], vbuf.at[slot],
                                       vcp_sems.at[slot])
            kc.start()
            vc.start()
            return kc, vc

        scale = D ** -0.5
        pend = [copy_in(0, 0), copy_in(1, 1)]
        o_rows = []
        l_cols = []
        for b in range(B):
            slot = b % 2
            for cp in pend[0]:
                cp.wait()
            pend = pend[1:]
            qm = e8 * q_ref[b:b + 1, :]
            s_t = lax.dot_general(
                qm, kbuf[slot], (((1,), (1,)), ((), ()))
            )
            p_t = jnp.exp(s_t * scale)
            cross = jax.lax.dot(p_t, vbuf[slot])
            o_rows.append(jnp.sum(cross * e8, axis=0, keepdims=True))
            l_cols.append(jnp.sum(p_t, axis=1, keepdims=True))
            if b + 2 < B:
                pend.append(copy_in(b + 2, slot))
        o8 = jnp.concatenate(o_rows, axis=0)
        l_hb = jnp.concatenate(l_cols, axis=1)
        l_bh = lax.dot_general(l_hb, eye8, (((0,), (0,)), ((), ())))
        comm[0] = jnp.concatenate(
            [o8, l_bh, jnp.zeros((B, PACK - HD - H), jnp.float32)], axis=1
        )

        pl.semaphore_wait(barrier_sem, Z_DEV - 1)

        sends = []
        for r in (1, 2, 3):
            send = pltpu.make_async_remote_copy(
                src_ref=comm.at[0],
                dst_ref=comm.at[Z_DEV - r],
                send_sem=send_sems.at[r - 1],
                recv_sem=recv_sems.at[Z_DEV - r - 1],
                device_id=(my_x, my_y, (my_z + r) % Z_DEV),
                device_id_type=pl.DeviceIdType.MESH,
            )
            send.start()
            sends.append(send)
        for t in (1, 2, 3):
            recv = pltpu.make_async_remote_copy(
                src_ref=comm.at[0],
                dst_ref=comm.at[t],
                send_sem=send_sems.at[t - 1],
                recv_sem=recv_sems.at[t - 1],
                device_id=(my_x, my_y, my_z),
                device_id_type=pl.DeviceIdType.MESH,
            )
            recv.wait_recv()

        tot = jnp.sum(comm[...], axis=0)
        o_sum = tot[:, :HD]
        l_sum = tot[:, HD:HD + H]
        l_flat = jax.lax.dot(l_sum, e8)
        out_ref[...] = o_sum / l_flat

        for send in sends:
            send.wait_send()

    out = pl.pallas_call(
        body,
        out_shape=jax.ShapeDtypeStruct((B, HD), jnp.float32),
        in_specs=[
            pl.BlockSpec(memory_space=pltpu.VMEM),
            pl.BlockSpec(memory_space=pl.ANY),
            pl.BlockSpec(memory_space=pl.ANY),
        ],
        out_specs=pl.BlockSpec(memory_space=pltpu.VMEM),
        scratch_shapes=[
            pltpu.VMEM((2, SKV, HD), jnp.float32),
            pltpu.VMEM((2, SKV, HD), jnp.float32),
            pltpu.VMEM((Z_DEV, B, PACK), jnp.float32),
            pltpu.SemaphoreType.DMA((2,)),
            pltpu.SemaphoreType.DMA((2,)),
            pltpu.SemaphoreType.DMA((Z_DEV - 1,)),
            pltpu.SemaphoreType.DMA((Z_DEV - 1,)),
        ],
        compiler_params=pltpu.CompilerParams(collective_id=0),
    )(q2, k2, v2)
    return out.reshape(B, SQ, H, D)
